# baseline (speedup 1.0000x reference)
import functools

import jax
import jax.numpy as jnp
from jax import lax
from jax.experimental import pallas as pl
from jax.experimental.pallas import tpu as pltpu
from jax.experimental.pallas import tpu_sc as plsc


def _make_sc_dots(B, NEG, D, V):
    info = plsc.get_sparse_core_info()
    NC, NS, L = info.num_cores, info.num_subcores, info.num_lanes
    NW = NC * NS
    bpw = B // NW
    RW = bpw * L // 128

    mesh = plsc.VectorSubcoreMesh(core_axis_name="c", subcore_axis_name="s")

    @functools.partial(
        pl.kernel,
        mesh=mesh,
        out_type=(
            jax.ShapeDtypeStruct((B, L), jnp.float32),
            jax.ShapeDtypeStruct((NEG, B * L // 128, 128), jnp.float32),
        ),
        scratch_types=[
            pltpu.VMEM((bpw,), jnp.int32),
            pltpu.VMEM((bpw,), jnp.int32),
            pltpu.VMEM((NEG, bpw), jnp.int32),
            pltpu.VMEM((bpw, 128), jnp.float32),
            pltpu.VMEM((bpw, 128), jnp.float32),
            pltpu.VMEM((bpw // 2, 128), jnp.float32),
            pltpu.VMEM((bpw // 2, 128), jnp.float32),
            pltpu.VMEM((bpw // 2, 128), jnp.float32),
            pltpu.VMEM((bpw // 2, 128), jnp.float32),
            pltpu.VMEM((bpw, L), jnp.float32),
            pltpu.VMEM((NEG, RW, 128), jnp.float32),
            pltpu.SemaphoreType.DMA,
            pltpu.SemaphoreType.DMA,
            pltpu.SemaphoreType.DMA,
            pltpu.SemaphoreType.DMA,
            pltpu.SemaphoreType.DMA,
            pltpu.SemaphoreType.DMA,
        ],
    )
    def sc_dots(cw, ctw, negT, tab, corr_out, negd_out,
                idx_v, idx_v2, idxn_v, rows_c, rows_x, rows_n0, rows_n1,
                rows_n2, rows_n3, corr_v, negd_v,
                semA, semB, sem0, sem1, sem2, sem3):
        wid = lax.axis_index("s") * NC + lax.axis_index("c")
        base = wid * bpw

        pltpu.sync_copy(cw.at[pl.ds(base, bpw)], idx_v)
        cpc = pltpu.async_copy(tab.at[idx_v], rows_c, semA)
        pltpu.sync_copy(ctw.at[pl.ds(base, bpw)], idx_v2)
        cpx = pltpu.async_copy(tab.at[idx_v2], rows_x, semB)
        pltpu.sync_copy(negT.at[:, pl.ds(base, bpw)], idxn_v)

        H = bpw // 2
        nbufs = (rows_n0, rows_n1, rows_n2, rows_n3)
        nsems = (sem0, sem1, sem2, sem3)

        def fire(hc, b):
            idx = idxn_v.at[hc // 2, pl.ds((hc % 2) * H, H)]
            pltpu.async_copy(tab.at[idx], nbufs[b], nsems[b])

        for b in range(4):
            fire(b, b)

        def sample_partial(ra, oa, ia, rb, ob, ib):
            acc = None
            for j in range(D // L):
                pa = (ra[ia, pl.ds(oa + j * L, L)] *
                      rb[ib, pl.ds(ob + j * L, L)])
                acc = pa if acc is None else acc + pa
            return acc

        cpc.wait()
        cpx.wait()

        def corr_body(g, carry):
            for u in range(L):
                i = g * L + u
                corr_v[i, :] = sample_partial(rows_c, 0, i, rows_x, D, i)
            return carry

        lax.fori_loop(0, bpw // L, corr_body, 0)

        NHC = 2 * NEG

        def qq_body(qq, carry):
            for b in range(4):
                hc = 4 * qq + b
                k = hc // 2
                base_i = (hc % 2) * H
                buf = nbufs[b]
                pltpu.make_async_copy(tab.at[idxn_v.at[0, pl.ds(0, H)]],
                                      buf, nsems[b]).wait()

                def gbody(g, c2, k=k, base_i=base_i, buf=buf):
                    for u in range(L):
                        j = g * L + u
                        i = base_i + j
                        p = sample_partial(buf, D, j, rows_c, 0, i)
                        negd_v[k, i // 8, pl.ds((i % 8) * L, L)] = p
                    return c2

                lax.fori_loop(0, H // L, gbody, 0)

                @pl.when(hc + 4 < NHC)
                def _(hc=hc, b=b):
                    fire(hc + 4, b)

            return carry

        lax.fori_loop(0, NHC // 4, qq_body, 0)

        pltpu.sync_copy(corr_v, corr_out.at[pl.ds(base, bpw), :])
        pltpu.sync_copy(negd_v, negd_out.at[:, pl.ds(wid * RW, RW), :])

    return sc_dots


def _make_tc_pack(V, D, VB=12800):
    grid = pl.cdiv(V, VB)

    def body(ct_ref, xt_ref, out_ref):
        eye = (lax.broadcasted_iota(jnp.int32, (D, D), 0) ==
               lax.broadcasted_iota(jnp.int32, (D, D), 1)).astype(jnp.float32)
        dn = (((0,), (0,)), ((), ()))
        ct = lax.dot_general(ct_ref[:, :], eye, dn,
                             preferred_element_type=jnp.float32)
        xt = lax.dot_general(xt_ref[:, :], eye, dn,
                             preferred_element_type=jnp.float32)
        out_ref[:, 0:D] = ct
        out_ref[:, D:2 * D] = xt

    return pl.pallas_call(
        body,
        grid=(grid,),
        in_specs=[
            pl.BlockSpec((D, VB), lambda i: (0, i)),
            pl.BlockSpec((D, VB), lambda i: (0, i)),
        ],
        out_specs=pl.BlockSpec((VB, 2 * D), lambda i: (i, 0)),
        out_shape=jax.ShapeDtypeStruct((V, 2 * D), jnp.float32),
    )


def _logsig(x):
    return jnp.minimum(x, 0.0) - jnp.log1p(jnp.exp(-jnp.abs(x)))


def _make_tc_broadcast(B, NEG, L, TIL=512):
    grid = B // TIL
    R = B * L // 128

    def body(corr_hbm, negd_hbm, out_ref, corr_v, negd_v, a_s, b_s, sem):
        t = pl.program_id(0)

        @pl.when(t == 0)
        def _():
            cp1 = pltpu.make_async_copy(corr_hbm, corr_v, sem)
            cp1.start()
            cp1.wait()
            cp2 = pltpu.make_async_copy(negd_hbm, negd_v, sem)
            cp2.start()
            cp2.wait()

            ones_l = jnp.ones((L, 1), jnp.float32)
            cd = jnp.dot(corr_v[:, :], ones_l,
                         preferred_element_type=jnp.float32)
            a_s[:, :] = _logsig(cd)

            fold = (lax.broadcasted_iota(jnp.int32, (128, 8), 0) // L ==
                    lax.broadcasted_iota(jnp.int32, (128, 8), 1)
                    ).astype(jnp.float32)
            nd = jnp.dot(jnp.reshape(negd_v[:, :, :], (NEG * R, 128)), fold,
                         preferred_element_type=jnp.float32)
            nl = _logsig(nd)
            b8 = nl[0:R, :]
            for k in range(1, NEG):
                b8 = b8 + nl[k * R:(k + 1) * R, :]
            e8 = (lax.broadcasted_iota(jnp.int32, (8, 128), 1) % 8 ==
                  lax.broadcasted_iota(jnp.int32, (8, 128), 0)
                  ).astype(jnp.float32)
            bex = jnp.dot(b8, e8,
                          preferred_element_type=jnp.float32)
            lmask = (lax.broadcasted_iota(jnp.int32, (R, 128), 1) // 8 ==
                     lax.broadcasted_iota(jnp.int32, (R, 128), 0) % L
                     ).astype(jnp.float32)
            mq = (lax.broadcasted_iota(jnp.int32, (32, R), 1) // L ==
                  lax.broadcasted_iota(jnp.int32, (32, R), 0)
                  ).astype(jnp.float32)
            b32 = jnp.dot(mq, bex * lmask,
                          preferred_element_type=jnp.float32)
            b_s[:, :, :] = b32[None, :, :]

        a_blk = a_s[pl.ds(t * TIL, TIL), :]
        out_ref[:, :, :] = -(a_blk[:, :, None] + b_s[:, :, :])

    return pl.pallas_call(
        body,
        grid=(grid,),
        in_specs=[
            pl.BlockSpec(memory_space=pl.ANY),
            pl.BlockSpec(memory_space=pl.ANY),
        ],
        out_specs=pl.BlockSpec((TIL, 32, 128), lambda i: (i, 0, 0)),
        out_shape=jax.ShapeDtypeStruct((B, 32, 128), jnp.float32),
        scratch_shapes=[
            pltpu.VMEM((B, L), jnp.float32),
            pltpu.VMEM((NEG, R, 128), jnp.float32),
            pltpu.VMEM((B, 1), jnp.float32),
            pltpu.VMEM((1, 32, 128), jnp.float32),
            pltpu.SemaphoreType.DMA,
        ],
    )


def kernel(center_word, context_word, neg_samples, center_table, context_table):
    B = center_word.shape[0]
    NEG = neg_samples.shape[1]
    V, D = center_table.shape
    L = 16

    tab = _make_tc_pack(V, D)(center_table.T, context_table.T)

    cw = center_word.astype(jnp.int32)
    ctw = context_word.astype(jnp.int32)
    neg_t = neg_samples.T.astype(jnp.int32)

    sc = _make_sc_dots(B, NEG, D, V)
    corr_p, negd_p = sc(cw, ctw, neg_t, tab)

    tc = _make_tc_broadcast(B, NEG, L)
    out = tc(corr_p, negd_p)
    return jnp.reshape(out, (B, B, 1))


# speedup vs baseline: 1.0804x; 1.0804x over previous
import functools

import jax
import jax.numpy as jnp
from jax import lax
from jax.experimental import pallas as pl
from jax.experimental.pallas import tpu as pltpu
from jax.experimental.pallas import tpu_sc as plsc


def _make_sc_dots(B, NEG, D, V):
    info = plsc.get_sparse_core_info()
    NC, NS, L = info.num_cores, info.num_subcores, info.num_lanes
    NW = NC * NS
    bpw = B // NW
    RW = bpw * L // 128

    mesh = plsc.VectorSubcoreMesh(core_axis_name="c", subcore_axis_name="s")

    @functools.partial(
        pl.kernel,
        mesh=mesh,
        out_type=(
            jax.ShapeDtypeStruct((B, L), jnp.float32),
            jax.ShapeDtypeStruct((NEG, B * L // 128, 128), jnp.float32),
        ),
        scratch_types=[
            pltpu.VMEM((bpw,), jnp.int32),
            pltpu.VMEM((bpw,), jnp.int32),
            pltpu.VMEM((NEG, bpw), jnp.int32),
            pltpu.VMEM((bpw, 128), jnp.float32),
            pltpu.VMEM((bpw, 128), jnp.float32),
            pltpu.VMEM((bpw, 128), jnp.float32),
            pltpu.VMEM((bpw, 128), jnp.float32),
            pltpu.VMEM((bpw, L), jnp.float32),
            pltpu.VMEM((NEG, RW, 128), jnp.float32),
            pltpu.SemaphoreType.DMA,
            pltpu.SemaphoreType.DMA,
            pltpu.SemaphoreType.DMA,
            pltpu.SemaphoreType.DMA,
        ],
    )
    def sc_dots(cw, ctw, negT, tab, corr_out, negd_out,
                idx_v, idx_v2, idxn_v, rows_c, rows_x, rows_n0, rows_n1,
                corr_v, negd_v, semA, semB, sem0, sem1):
        wid = lax.axis_index("s") * NC + lax.axis_index("c")
        base = wid * bpw

        pltpu.sync_copy(cw.at[pl.ds(base, bpw)], idx_v)
        cpc = pltpu.async_copy(tab.at[idx_v], rows_c, semA)
        pltpu.sync_copy(ctw.at[pl.ds(base, bpw)], idx_v2)
        cpx = pltpu.async_copy(tab.at[idx_v2], rows_x, semB)
        pltpu.sync_copy(negT.at[:, pl.ds(base, bpw)], idxn_v)

        nbufs = (rows_n0, rows_n1)
        nsems = (sem0, sem1)
        pltpu.async_copy(tab.at[idxn_v.at[0]], rows_n0, sem0)
        pltpu.async_copy(tab.at[idxn_v.at[1]], rows_n1, sem1)

        def sample_partial(ra, oa, ia, rb, ob, ib):
            acc = None
            for j in range(D // L):
                pa = (ra[ia, pl.ds(oa + j * L, L)] *
                      rb[ib, pl.ds(ob + j * L, L)])
                acc = pa if acc is None else acc + pa
            return acc

        cpc.wait()
        cpx.wait()

        def corr_body(g, carry):
            for u in range(L):
                i = g * L + u
                corr_v[i, :] = sample_partial(rows_c, 0, i, rows_x, D, i)
            return carry

        lax.fori_loop(0, bpw // L, corr_body, 0)

        def kk_body(kk, carry):
            for b in range(2):
                k = 2 * kk + b
                buf = nbufs[b]
                pltpu.make_async_copy(tab.at[idxn_v.at[0]], buf,
                                      nsems[b]).wait()

                def gbody(g, c2, k=k, buf=buf):
                    for u in range(L):
                        i = g * L + u
                        p = sample_partial(buf, D, i, rows_c, 0, i)
                        negd_v[k, i // 8, pl.ds((i % 8) * L, L)] = p
                    return c2

                lax.fori_loop(0, bpw // L, gbody, 0)

                @pl.when(k + 2 < NEG)
                def _(k=k, b=b, buf=buf):
                    pltpu.async_copy(tab.at[idxn_v.at[k + 2]], buf, nsems[b])

            return carry

        lax.fori_loop(0, NEG // 2, kk_body, 0)

        pltpu.sync_copy(corr_v, corr_out.at[pl.ds(base, bpw), :])
        pltpu.sync_copy(negd_v, negd_out.at[:, pl.ds(wid * RW, RW), :])

    return sc_dots


def _make_tc_pack(V, D, VB=12800):
    grid = pl.cdiv(V, VB)

    def body(ct_ref, xt_ref, out_ref):
        eye = (lax.broadcasted_iota(jnp.int32, (D, D), 0) ==
               lax.broadcasted_iota(jnp.int32, (D, D), 1)).astype(jnp.float32)
        dn = (((0,), (0,)), ((), ()))
        ct = lax.dot_general(ct_ref[:, :], eye, dn,
                             preferred_element_type=jnp.float32)
        xt = lax.dot_general(xt_ref[:, :], eye, dn,
                             preferred_element_type=jnp.float32)
        out_ref[:, 0:D] = ct
        out_ref[:, D:2 * D] = xt

    return pl.pallas_call(
        body,
        grid=(grid,),
        in_specs=[
            pl.BlockSpec((D, VB), lambda i: (0, i)),
            pl.BlockSpec((D, VB), lambda i: (0, i)),
        ],
        out_specs=pl.BlockSpec((VB, 2 * D), lambda i: (i, 0)),
        out_shape=jax.ShapeDtypeStruct((V, 2 * D), jnp.float32),
    )


def _logsig(x):
    return jnp.minimum(x, 0.0) - jnp.log1p(jnp.exp(-jnp.abs(x)))


def _make_tc_broadcast(B, NEG, L, TIL=512):
    grid = B // TIL
    R = B * L // 128

    def body(corr_hbm, negd_hbm, out_ref, corr_v, negd_v, a_s, b_s, sem):
        t = pl.program_id(0)

        @pl.when(t == 0)
        def _():
            cp1 = pltpu.make_async_copy(corr_hbm, corr_v, sem)
            cp1.start()
            cp1.wait()
            cp2 = pltpu.make_async_copy(negd_hbm, negd_v, sem)
            cp2.start()
            cp2.wait()

            ones_l = jnp.ones((L, 1), jnp.float32)
            cd = jnp.dot(corr_v[:, :], ones_l,
                         preferred_element_type=jnp.float32)
            a_s[:, :] = _logsig(cd)

            fold = (lax.broadcasted_iota(jnp.int32, (128, 8), 0) // L ==
                    lax.broadcasted_iota(jnp.int32, (128, 8), 1)
                    ).astype(jnp.float32)
            nd = jnp.dot(jnp.reshape(negd_v[:, :, :], (NEG * R, 128)), fold,
                         preferred_element_type=jnp.float32)
            nl = _logsig(nd)
            b8 = nl[0:R, :]
            for k in range(1, NEG):
                b8 = b8 + nl[k * R:(k + 1) * R, :]
            e8 = (lax.broadcasted_iota(jnp.int32, (8, 128), 1) % 8 ==
                  lax.broadcasted_iota(jnp.int32, (8, 128), 0)
                  ).astype(jnp.float32)
            bex = jnp.dot(b8, e8,
                          preferred_element_type=jnp.float32)
            lmask = (lax.broadcasted_iota(jnp.int32, (R, 128), 1) // 8 ==
                     lax.broadcasted_iota(jnp.int32, (R, 128), 0) % L
                     ).astype(jnp.float32)
            mq = (lax.broadcasted_iota(jnp.int32, (32, R), 1) // L ==
                  lax.broadcasted_iota(jnp.int32, (32, R), 0)
                  ).astype(jnp.float32)
            b32 = jnp.dot(mq, bex * lmask,
                          preferred_element_type=jnp.float32)
            b_s[:, :, :] = b32[None, :, :]

        a_blk = a_s[pl.ds(t * TIL, TIL), :]
        out_ref[:, :, :] = -(a_blk[:, :, None] + b_s[:, :, :])

    return pl.pallas_call(
        body,
        grid=(grid,),
        in_specs=[
            pl.BlockSpec(memory_space=pl.ANY),
            pl.BlockSpec(memory_space=pl.ANY),
        ],
        out_specs=pl.BlockSpec((TIL, 32, 128), lambda i: (i, 0, 0)),
        out_shape=jax.ShapeDtypeStruct((B, 32, 128), jnp.float32),
        scratch_shapes=[
            pltpu.VMEM((B, L), jnp.float32),
            pltpu.VMEM((NEG, R, 128), jnp.float32),
            pltpu.VMEM((B, 1), jnp.float32),
            pltpu.VMEM((1, 32, 128), jnp.float32),
            pltpu.SemaphoreType.DMA,
        ],
    )


def kernel(center_word, context_word, neg_samples, center_table, context_table):
    B = center_word.shape[0]
    NEG = neg_samples.shape[1]
    V, D = center_table.shape
    L = 16

    tab = _make_tc_pack(V, D)(center_table.T, context_table.T)

    cw = center_word.astype(jnp.int32)
    ctw = context_word.astype(jnp.int32)
    neg_t = neg_samples.T.astype(jnp.int32)

    sc = _make_sc_dots(B, NEG, D, V)
    corr_p, negd_p = sc(cw, ctw, neg_t, tab)

    tc = _make_tc_broadcast(B, NEG, L)
    out = tc(corr_p, negd_p)
    return jnp.reshape(out, (B, B, 1))
